# baseline (speedup 1.0000x reference)
import functools

import jax
import jax.numpy as jnp
from jax import lax
from jax.experimental import pallas as pl
from jax.experimental.pallas import tpu as pltpu
from jax.experimental.pallas import tpu_sc as plsc

DIM = 100000
RANK = 64
BATCH = 16384
NC = 2
NS = 16
NW = NC * NS
BPW = BATCH // NW
CHUNK = 128
NCHUNK = BPW // CHUNK


def _sc_body(i1_hbm, i2_hbm, ut_hbm, vt_hbm, bu_hbm, bv_hbm, out_hbm,
             idx1_v, idx2_v, u_v, v_v, b1_v, b2_v, acc_v, sem):
    wid = lax.axis_index("s") * NC + lax.axis_index("c")
    base = wid * BPW

    pltpu.sync_copy(i1_hbm.at[wid], idx1_v)
    pltpu.sync_copy(i2_hbm.at[wid], idx2_v)

    copies = []
    for j in range(NCHUNK):
        off = j * CHUNK
        copies.append(pltpu.async_copy(
            ut_hbm.at[idx1_v.at[j]], u_v.at[pl.ds(off, CHUNK)], sem))
        copies.append(pltpu.async_copy(
            vt_hbm.at[idx2_v.at[j]], v_v.at[pl.ds(off, CHUNK)], sem))
        copies.append(pltpu.async_copy(
            bu_hbm.at[idx1_v.at[j]], b1_v.at[pl.ds(off, CHUNK)], sem))
        copies.append(pltpu.async_copy(
            bv_hbm.at[idx2_v.at[j]], b2_v.at[pl.ds(off, CHUNK)], sem))
    for c in copies:
        c.wait()

    def cbody(c, carry):
        o16 = c * 16
        rows = o16 + lax.iota(jnp.int32, 16)
        acc = b1_v[pl.ds(o16, 16)] + b2_v[pl.ds(o16, 16)]
        for j in range(RANK):
            cj = jnp.full((16,), j, jnp.int32)
            acc = acc + (plsc.load_gather(u_v, [rows, cj])
                         * plsc.load_gather(v_v, [rows, cj]))
        acc_v[pl.ds(o16, 16)] = acc
        return carry

    lax.fori_loop(0, BPW // 16, cbody, 0)
    pltpu.sync_copy(acc_v, out_hbm.at[pl.ds(base, BPW)])


@functools.partial(
    pl.kernel,
    out_type=jax.ShapeDtypeStruct((BATCH,), jnp.float32),
    mesh=plsc.VectorSubcoreMesh(core_axis_name="c", subcore_axis_name="s"),
    compiler_params=pltpu.CompilerParams(
        needs_layout_passes=False, use_tc_tiling_on_sc=False),
    scratch_types=[
        pltpu.VMEM((NCHUNK, CHUNK), jnp.int32),
        pltpu.VMEM((NCHUNK, CHUNK), jnp.int32),
        pltpu.VMEM((BPW, RANK), jnp.float32),
        pltpu.VMEM((BPW, RANK), jnp.float32),
        pltpu.VMEM((BPW,), jnp.float32),
        pltpu.VMEM((BPW,), jnp.float32),
        pltpu.VMEM((BPW,), jnp.float32),
        pltpu.SemaphoreType.DMA,
    ],
)
def _sc_kernel(i1_hbm, i2_hbm, ut_hbm, vt_hbm, bu_hbm, bv_hbm, out_hbm,
               *scratch):
    _sc_body(i1_hbm, i2_hbm, ut_hbm, vt_hbm, bu_hbm, bv_hbm, out_hbm,
             *scratch)


def kernel(x, U_w, V_w, bias_U, bias_V):
    i1 = x[:, 0].astype(jnp.int32).reshape(NW, NCHUNK, CHUNK)
    i2 = x[:, 1].astype(jnp.int32).reshape(NW, NCHUNK, CHUNK)
    ut = U_w.T
    vt = V_w.T
    out = _sc_kernel(i1, i2, ut, vt, bias_U, bias_V)
    return out[:, None]
